# baseline (speedup 1.0000x reference)
import functools

import jax
import jax.numpy as jnp
from jax import lax
from jax.experimental import pallas as pl
from jax.experimental.pallas import tpu as pltpu
from jax.experimental.pallas import tpu_sc as plsc

ROWS = 576
VOCAB = 100000
NEG_INF = float("-inf")
BIG_I32 = 2**31 - 1

SC_ROWS = 256
TC_ROWS = ROWS - SC_ROWS
CHUNK = 4096
NCHUNK = (VOCAB + CHUNK - 1) // CHUNK

NW = 32
RPW = SC_ROWS // NW
SCCH = 6400
NCH_SC = 14
SC_COLS = NCH_SC * SCCH
ITERS_U = SCCH // 16

TC2_RB = 64
TC2_NC = (VOCAB - SC_COLS + SCCH - 1) // SCCH


def _tc_argmax_body(x_ref, outv_ref, outc_ref, best_val, best_chunk):
    i = pl.program_id(0)

    @pl.when(i == 0)
    def _():
        best_val[...] = jnp.full((TC_ROWS, 1), NEG_INF, jnp.float32)
        best_chunk[...] = jnp.zeros((TC_ROWS, 1), jnp.int32)

    def _update(xm):
        m = jnp.max(xm, axis=1, keepdims=True)
        better = m > best_val[...]
        best_val[...] = jnp.where(better, m, best_val[...])
        best_chunk[...] = jnp.where(better, i, best_chunk[...])

    @pl.when(i < NCHUNK - 1)
    def _():
        _update(x_ref[...])

    @pl.when(i == NCHUNK - 1)
    def _():
        col = jax.lax.broadcasted_iota(
            jnp.int32, (TC_ROWS, CHUNK), 1) + i * CHUNK
        _update(jnp.where(col < VOCAB, x_ref[...], NEG_INF))
        outv_ref[...] = best_val[...]
        outc_ref[...] = best_chunk[...]


def _rescan_body(bc_ref, x_ref, bv_ref, out_ref):
    r = pl.program_id(0)
    base = bc_ref[r] * CHUNK
    j = lax.rem(r, 8)
    x = x_ref[pl.ds(j, 1), :]
    bv = bv_ref[pl.ds(j, 1), :]
    col = jax.lax.broadcasted_iota(jnp.int32, (1, CHUNK), 1)
    eq = (x == bv) & (col + base < VOCAB)
    out_ref[pl.ds(j, 1), :] = jnp.min(
        jnp.where(eq, col, BIG_I32), axis=1, keepdims=True) + base


@functools.partial(
    pl.kernel,
    out_type=(jax.ShapeDtypeStruct((SC_ROWS * 16,), jnp.int32),
              jax.ShapeDtypeStruct((SC_ROWS * 16,), jnp.float32)),
    mesh=plsc.VectorSubcoreMesh(core_axis_name="c", subcore_axis_name="s"),
    scratch_types=[
        pltpu.VMEM((RPW, SCCH), jnp.float32),
        pltpu.VMEM((RPW, SCCH), jnp.float32),
        pltpu.VMEM((RPW * 16,), jnp.float32),
        pltpu.VMEM((RPW * 16,), jnp.int32),
        pltpu.VMEM((RPW * 16,), jnp.int32),
        pltpu.VMEM((RPW * 16,), jnp.float32),
        pltpu.SemaphoreType.DMA,
        pltpu.SemaphoreType.DMA,
    ],
)
def _sc_argmax(tp_hbm, outc_hbm, outv_hbm, buf0, buf1,
               bv_ref, bj_ref, resc_ref, resv_ref, sem0, sem1):
    wid = lax.axis_index("s") * 2 + lax.axis_index("c")
    lane = lax.iota(jnp.int32, 16)
    row0 = TC_ROWS + wid * RPW

    def init_row(r, _):
        bv_ref[pl.ds(r * 16, 16)] = jnp.full((16,), NEG_INF, jnp.float32)
        bj_ref[pl.ds(r * 16, 16)] = jnp.zeros((16,), jnp.int32)
        return 0

    lax.fori_loop(0, RPW, init_row, 0)

    def start(chunk, buf, sem):
        off = pl.multiple_of(chunk * SCCH, 128)
        return pltpu.async_copy(
            tp_hbm.at[pl.ds(row0, RPW), pl.ds(off, SCCH)], buf, sem)

    def wait(buf, sem):
        pltpu.make_async_copy(
            tp_hbm.at[pl.ds(row0, RPW), pl.ds(0, SCCH)], buf, sem).wait()

    def scan_chunk(buf, base, iters, unroll):
        def row_body(r, _):
            pv = bv_ref[pl.ds(r * 16, 16)]
            pj = bj_ref[pl.ds(r * 16, 16)]

            def body(j, carry):
                pv, pj = carry
                for u in range(unroll):
                    x = buf[r, pl.ds(j * (unroll * 16) + u * 16, 16)]
                    better = x > pv
                    pv = jnp.where(better, x, pv)
                    pj = jnp.where(
                        better,
                        jnp.full((16,), base + j * unroll + u, jnp.int32), pj)
                return pv, pj

            pv, pj = lax.fori_loop(0, iters // unroll, body, (pv, pj))
            bv_ref[pl.ds(r * 16, 16)] = pv
            bj_ref[pl.ds(r * 16, 16)] = pj
            return 0

        lax.fori_loop(0, RPW, row_body, 0)

    start(0, buf0, sem0)
    start(1, buf1, sem1)

    def pair_body(p, _):
        k = 2 * p
        wait(buf0, sem0)
        scan_chunk(buf0, k * ITERS_U, ITERS_U, 16)

        @pl.when(p < NCH_SC // 2 - 1)
        def _():
            start(k + 2, buf0, sem0)

        wait(buf1, sem1)
        scan_chunk(buf1, (k + 1) * ITERS_U, ITERS_U, 16)

        @pl.when(p < NCH_SC // 2 - 1)
        def _():
            start(k + 3, buf1, sem1)

        return 0

    lax.fori_loop(0, NCH_SC // 2, pair_body, 0)

    for r in range(RPW):
        resc_ref[pl.ds(r * 16, 16)] = bj_ref[pl.ds(r * 16, 16)] * 16 + lane
        resv_ref[pl.ds(r * 16, 16)] = bv_ref[pl.ds(r * 16, 16)]
    pltpu.sync_copy(resc_ref, outc_hbm.at[pl.ds(wid * RPW * 16, RPW * 16)])
    pltpu.sync_copy(resv_ref, outv_hbm.at[pl.ds(wid * RPW * 16, RPW * 16)])


def _tc2_body(x_ref, outv_ref, outi_ref, best_val, best_idx):
    c = pl.program_id(1)

    @pl.when(c == 0)
    def _():
        best_val[...] = jnp.full((TC2_RB, 1), NEG_INF, jnp.float32)
        best_idx[...] = jnp.zeros((TC2_RB, 1), jnp.int32)

    col = (jax.lax.broadcasted_iota(jnp.int32, (TC2_RB, SCCH), 1)
           + SC_COLS + c * SCCH)
    xm = jnp.where(col < VOCAB, x_ref[...], NEG_INF)
    m = jnp.max(xm, axis=1, keepdims=True)
    idx = jnp.min(jnp.where(xm == m, col, BIG_I32), axis=1, keepdims=True)
    better = m > best_val[...]
    best_val[...] = jnp.where(better, m, best_val[...])
    best_idx[...] = jnp.where(better, idx, best_idx[...])

    @pl.when(c == TC2_NC - 1)
    def _():
        outv_ref[...] = best_val[...]
        outi_ref[...] = best_idx[...]


def _finish_body(vals_ref, cols_ref, tc2v_ref, tc2i_ref, tc_idx_ref,
                 draft_ref, out_ref):
    v = vals_ref[...]
    m = jnp.max(v, axis=1, keepdims=True)
    sci = jnp.min(jnp.where(v == m, cols_ref[...], BIG_I32),
                  axis=1, keepdims=True)
    better = tc2v_ref[...] > m
    merged = jnp.where(better, tc2i_ref[...], sci)

    ids = jnp.concatenate(
        [tc_idx_ref[...], merged], axis=0).reshape(64, 9)
    draft = draft_ref[...]
    match = ids[:, :8] == draft
    j = jax.lax.broadcasted_iota(jnp.int32, (64, 8), 1)
    n = jnp.min(jnp.where(match, jnp.int32(8), j), axis=1, keepdims=True)
    p = jax.lax.broadcasted_iota(jnp.int32, (64, 9), 1)
    out_ref[...] = jnp.where(p <= n, ids, jnp.int32(-1))


@jax.jit
def kernel(draft_token_ids, target_probs):
    tc_val, tc_chunk = pl.pallas_call(
        _tc_argmax_body,
        grid=(NCHUNK,),
        in_specs=[pl.BlockSpec((TC_ROWS, CHUNK), lambda i: (0, i))],
        out_specs=(pl.BlockSpec((TC_ROWS, 1), lambda i: (0, 0)),
                   pl.BlockSpec((TC_ROWS, 1), lambda i: (0, 0))),
        out_shape=(jax.ShapeDtypeStruct((TC_ROWS, 1), jnp.float32),
                   jax.ShapeDtypeStruct((TC_ROWS, 1), jnp.int32)),
        scratch_shapes=[
            pltpu.VMEM((TC_ROWS, 1), jnp.float32),
            pltpu.VMEM((TC_ROWS, 1), jnp.int32),
        ],
    )(target_probs)

    tc_idx = pl.pallas_call(
        _rescan_body,
        grid_spec=pltpu.PrefetchScalarGridSpec(
            num_scalar_prefetch=1,
            grid=(TC_ROWS,),
            in_specs=[
                pl.BlockSpec((8, CHUNK), lambda r, bc: (r // 8, bc[r])),
                pl.BlockSpec((8, 1), lambda r, bc: (r // 8, 0)),
            ],
            out_specs=pl.BlockSpec((8, 1), lambda r, bc: (r // 8, 0)),
        ),
        out_shape=jax.ShapeDtypeStruct((TC_ROWS, 1), jnp.int32),
    )(tc_chunk.reshape(TC_ROWS), target_probs, tc_val)

    tc2_val, tc2_idx = pl.pallas_call(
        _tc2_body,
        grid=(SC_ROWS // TC2_RB, TC2_NC),
        in_specs=[pl.BlockSpec(
            (TC2_RB, SCCH),
            lambda r, c: (r + TC_ROWS // TC2_RB, c + NCH_SC))],
        out_specs=(pl.BlockSpec((TC2_RB, 1), lambda r, c: (r, 0)),
                   pl.BlockSpec((TC2_RB, 1), lambda r, c: (r, 0))),
        out_shape=(jax.ShapeDtypeStruct((SC_ROWS, 1), jnp.float32),
                   jax.ShapeDtypeStruct((SC_ROWS, 1), jnp.int32)),
        scratch_shapes=[
            pltpu.VMEM((TC2_RB, 1), jnp.float32),
            pltpu.VMEM((TC2_RB, 1), jnp.int32),
        ],
    )(target_probs)

    sc_cols, sc_vals = _sc_argmax(target_probs)
    sc_cols = sc_cols.reshape(SC_ROWS, 16)
    sc_vals = sc_vals.reshape(SC_ROWS, 16)

    out = pl.pallas_call(
        _finish_body,
        grid=(1,),
        in_specs=[
            pl.BlockSpec((SC_ROWS, 16), lambda i: (0, 0)),
            pl.BlockSpec((SC_ROWS, 16), lambda i: (0, 0)),
            pl.BlockSpec((SC_ROWS, 1), lambda i: (0, 0)),
            pl.BlockSpec((SC_ROWS, 1), lambda i: (0, 0)),
            pl.BlockSpec((TC_ROWS, 1), lambda i: (0, 0)),
            pl.BlockSpec((64, 8), lambda i: (0, 0)),
        ],
        out_specs=pl.BlockSpec((64, 9), lambda i: (0, 0)),
        out_shape=jax.ShapeDtypeStruct((64, 9), jnp.int32),
    )(sc_vals, sc_cols, tc2_val, tc2_idx, tc_idx,
      draft_token_ids.astype(jnp.int32))
    return out.astype(jnp.int64)


# speedup vs baseline: 2.6300x; 2.6300x over previous
import functools

import jax
import jax.numpy as jnp
from jax import lax
from jax.experimental import pallas as pl
from jax.experimental.pallas import tpu as pltpu
from jax.experimental.pallas import tpu_sc as plsc

ROWS = 576
VOCAB = 100000
NEG_INF = float("-inf")
BIG_I32 = 2**31 - 1

SC_ROWS = 256
TC_ROWS = ROWS - SC_ROWS
CHUNK = 4096
NCHUNK = (VOCAB + CHUNK - 1) // CHUNK

NW = 32
RPW = SC_ROWS // NW
SCCH = 6400
NCH_SC = 14
SC_COLS = NCH_SC * SCCH
ITERS_U = SCCH // 16

TC2_RB = 64
TC2_NC = (VOCAB - SC_COLS + SCCH - 1) // SCCH


def _tc_argmax_body(x_ref, out_ref, acc_v, acc_c):
    i = pl.program_id(0)

    @pl.when(i == 0)
    def _():
        acc_v[...] = jnp.full((TC_ROWS, CHUNK), NEG_INF, jnp.float32)
        acc_c[...] = jnp.zeros((TC_ROWS, CHUNK), jnp.int32)

    def _update(x):
        better = x > acc_v[...]
        acc_v[...] = jnp.where(better, x, acc_v[...])
        acc_c[...] = jnp.where(better, i, acc_c[...])

    @pl.when(i < NCHUNK - 1)
    def _():
        _update(x_ref[...])

    @pl.when(i == NCHUNK - 1)
    def _():
        off = jax.lax.broadcasted_iota(jnp.int32, (TC_ROWS, CHUNK), 1)
        _update(jnp.where(off + i * CHUNK < VOCAB, x_ref[...], NEG_INF))
        v = acc_v[...]
        m = jnp.max(v, axis=1, keepdims=True)
        gcol = acc_c[...] * CHUNK + off
        out_ref[...] = jnp.min(
            jnp.where(v == m, gcol, BIG_I32), axis=1, keepdims=True)


@functools.partial(
    pl.kernel,
    out_type=(jax.ShapeDtypeStruct((SC_ROWS * 16,), jnp.int32),
              jax.ShapeDtypeStruct((SC_ROWS * 16,), jnp.float32)),
    mesh=plsc.VectorSubcoreMesh(core_axis_name="c", subcore_axis_name="s"),
    scratch_types=[
        pltpu.VMEM((RPW, SCCH), jnp.float32),
        pltpu.VMEM((RPW, SCCH), jnp.float32),
        pltpu.VMEM((RPW * 16,), jnp.float32),
        pltpu.VMEM((RPW * 16,), jnp.int32),
        pltpu.VMEM((RPW * 16,), jnp.int32),
        pltpu.VMEM((RPW * 16,), jnp.float32),
        pltpu.SemaphoreType.DMA,
        pltpu.SemaphoreType.DMA,
    ],
)
def _sc_argmax(tp_hbm, outc_hbm, outv_hbm, buf0, buf1,
               bv_ref, bj_ref, resc_ref, resv_ref, sem0, sem1):
    wid = lax.axis_index("s") * 2 + lax.axis_index("c")
    lane = lax.iota(jnp.int32, 16)
    row0 = TC_ROWS + wid * RPW

    def init_row(r, _):
        bv_ref[pl.ds(r * 16, 16)] = jnp.full((16,), NEG_INF, jnp.float32)
        bj_ref[pl.ds(r * 16, 16)] = jnp.zeros((16,), jnp.int32)
        return 0

    lax.fori_loop(0, RPW, init_row, 0)

    def start(chunk, buf, sem):
        off = pl.multiple_of(chunk * SCCH, 128)
        return pltpu.async_copy(
            tp_hbm.at[pl.ds(row0, RPW), pl.ds(off, SCCH)], buf, sem)

    def wait(buf, sem):
        pltpu.make_async_copy(
            tp_hbm.at[pl.ds(row0, RPW), pl.ds(0, SCCH)], buf, sem).wait()

    def scan_chunk(buf, base, iters, unroll):
        def row_body(r, _):
            pv = bv_ref[pl.ds(r * 16, 16)]
            pj = bj_ref[pl.ds(r * 16, 16)]

            def body(j, carry):
                pv, pj = carry
                for u in range(unroll):
                    x = buf[r, pl.ds(j * (unroll * 16) + u * 16, 16)]
                    better = x > pv
                    pv = jnp.where(better, x, pv)
                    pj = jnp.where(
                        better,
                        jnp.full((16,), base + j * unroll + u, jnp.int32), pj)
                return pv, pj

            pv, pj = lax.fori_loop(0, iters // unroll, body, (pv, pj))
            bv_ref[pl.ds(r * 16, 16)] = pv
            bj_ref[pl.ds(r * 16, 16)] = pj
            return 0

        lax.fori_loop(0, RPW, row_body, 0)

    start(0, buf0, sem0)
    start(1, buf1, sem1)

    def pair_body(p, _):
        k = 2 * p
        wait(buf0, sem0)
        scan_chunk(buf0, k * ITERS_U, ITERS_U, 16)

        @pl.when(p < NCH_SC // 2 - 1)
        def _():
            start(k + 2, buf0, sem0)

        wait(buf1, sem1)
        scan_chunk(buf1, (k + 1) * ITERS_U, ITERS_U, 16)

        @pl.when(p < NCH_SC // 2 - 1)
        def _():
            start(k + 3, buf1, sem1)

        return 0

    lax.fori_loop(0, NCH_SC // 2, pair_body, 0)

    for r in range(RPW):
        resc_ref[pl.ds(r * 16, 16)] = bj_ref[pl.ds(r * 16, 16)] * 16 + lane
        resv_ref[pl.ds(r * 16, 16)] = bv_ref[pl.ds(r * 16, 16)]
    pltpu.sync_copy(resc_ref, outc_hbm.at[pl.ds(wid * RPW * 16, RPW * 16)])
    pltpu.sync_copy(resv_ref, outv_hbm.at[pl.ds(wid * RPW * 16, RPW * 16)])


def _tc2_body(x_ref, outv_ref, outi_ref, best_val, best_idx):
    c = pl.program_id(1)

    @pl.when(c == 0)
    def _():
        best_val[...] = jnp.full((TC2_RB, 1), NEG_INF, jnp.float32)
        best_idx[...] = jnp.zeros((TC2_RB, 1), jnp.int32)

    col = (jax.lax.broadcasted_iota(jnp.int32, (TC2_RB, SCCH), 1)
           + SC_COLS + c * SCCH)
    xm = jnp.where(col < VOCAB, x_ref[...], NEG_INF)
    m = jnp.max(xm, axis=1, keepdims=True)
    idx = jnp.min(jnp.where(xm == m, col, BIG_I32), axis=1, keepdims=True)
    better = m > best_val[...]
    best_val[...] = jnp.where(better, m, best_val[...])
    best_idx[...] = jnp.where(better, idx, best_idx[...])

    @pl.when(c == TC2_NC - 1)
    def _():
        outv_ref[...] = best_val[...]
        outi_ref[...] = best_idx[...]


def _finish_body(vals_ref, cols_ref, tc2v_ref, tc2i_ref, tc_idx_ref,
                 draft_ref, out_ref):
    v = vals_ref[...]
    m = jnp.max(v, axis=1, keepdims=True)
    sci = jnp.min(jnp.where(v == m, cols_ref[...], BIG_I32),
                  axis=1, keepdims=True)
    better = tc2v_ref[...] > m
    merged = jnp.where(better, tc2i_ref[...], sci)

    ids = jnp.concatenate(
        [tc_idx_ref[...], merged], axis=0).reshape(64, 9)
    draft = draft_ref[...]
    match = ids[:, :8] == draft
    j = jax.lax.broadcasted_iota(jnp.int32, (64, 8), 1)
    n = jnp.min(jnp.where(match, jnp.int32(8), j), axis=1, keepdims=True)
    p = jax.lax.broadcasted_iota(jnp.int32, (64, 9), 1)
    out_ref[...] = jnp.where(p <= n, ids, jnp.int32(-1))


@jax.jit
def kernel(draft_token_ids, target_probs):
    tc_idx = pl.pallas_call(
        _tc_argmax_body,
        grid=(NCHUNK,),
        in_specs=[pl.BlockSpec((TC_ROWS, CHUNK), lambda i: (0, i))],
        out_specs=pl.BlockSpec((TC_ROWS, 1), lambda i: (0, 0)),
        out_shape=jax.ShapeDtypeStruct((TC_ROWS, 1), jnp.int32),
        scratch_shapes=[
            pltpu.VMEM((TC_ROWS, CHUNK), jnp.float32),
            pltpu.VMEM((TC_ROWS, CHUNK), jnp.int32),
        ],
    )(target_probs)

    tc2_val, tc2_idx = pl.pallas_call(
        _tc2_body,
        grid=(SC_ROWS // TC2_RB, TC2_NC),
        in_specs=[pl.BlockSpec(
            (TC2_RB, SCCH),
            lambda r, c: (r + TC_ROWS // TC2_RB, c + NCH_SC))],
        out_specs=(pl.BlockSpec((TC2_RB, 1), lambda r, c: (r, 0)),
                   pl.BlockSpec((TC2_RB, 1), lambda r, c: (r, 0))),
        out_shape=(jax.ShapeDtypeStruct((SC_ROWS, 1), jnp.float32),
                   jax.ShapeDtypeStruct((SC_ROWS, 1), jnp.int32)),
        scratch_shapes=[
            pltpu.VMEM((TC2_RB, 1), jnp.float32),
            pltpu.VMEM((TC2_RB, 1), jnp.int32),
        ],
    )(target_probs)

    sc_cols, sc_vals = _sc_argmax(target_probs)
    sc_cols = sc_cols.reshape(SC_ROWS, 16)
    sc_vals = sc_vals.reshape(SC_ROWS, 16)

    out = pl.pallas_call(
        _finish_body,
        grid=(1,),
        in_specs=[
            pl.BlockSpec((SC_ROWS, 16), lambda i: (0, 0)),
            pl.BlockSpec((SC_ROWS, 16), lambda i: (0, 0)),
            pl.BlockSpec((SC_ROWS, 1), lambda i: (0, 0)),
            pl.BlockSpec((SC_ROWS, 1), lambda i: (0, 0)),
            pl.BlockSpec((TC_ROWS, 1), lambda i: (0, 0)),
            pl.BlockSpec((64, 8), lambda i: (0, 0)),
        ],
        out_specs=pl.BlockSpec((64, 9), lambda i: (0, 0)),
        out_shape=jax.ShapeDtypeStruct((64, 9), jnp.int32),
    )(sc_vals, sc_cols, tc2_val, tc2_idx, tc_idx,
      draft_token_ids.astype(jnp.int32))
    return out.astype(jnp.int64)


# speedup vs baseline: 2.8428x; 1.0809x over previous
import functools

import jax
import jax.numpy as jnp
from jax import lax
from jax.experimental import pallas as pl
from jax.experimental.pallas import tpu as pltpu
from jax.experimental.pallas import tpu_sc as plsc

ROWS = 576
VOCAB = 100000
NEG_INF = float("-inf")
BIG_I32 = 2**31 - 1

SC_ROWS = 256
TC_ROWS = ROWS - SC_ROWS
CHUNK = 4096
NCHUNK = (VOCAB + CHUNK - 1) // CHUNK

NW = 32
RPW = 8
SCCH = 6400
NUNI = 15
SC_COLS = (VOCAB // 128) * 128
SCTAIL = SC_COLS - NUNI * SCCH
ITERS_U = SCCH // 16
ITERS_T = SCTAIL // 16


def _tc_argmax_body(x_ref, out_ref, best_val, best_idx):
    i = pl.program_id(0)

    @pl.when(i == 0)
    def _():
        best_val[...] = jnp.full((TC_ROWS, 1), NEG_INF, jnp.float32)
        best_idx[...] = jnp.zeros((TC_ROWS, 1), jnp.int32)

    def _update(xm, col):
        m = jnp.max(xm, axis=1, keepdims=True)
        idx = jnp.min(jnp.where(xm == m, col, BIG_I32), axis=1, keepdims=True)
        better = m > best_val[...]
        best_val[...] = jnp.where(better, m, best_val[...])
        best_idx[...] = jnp.where(better, idx, best_idx[...])

    col = jax.lax.broadcasted_iota(jnp.int32, (TC_ROWS, CHUNK), 1) + i * CHUNK

    @pl.when(i < NCHUNK - 1)
    def _():
        _update(x_ref[...], col)

    @pl.when(i == NCHUNK - 1)
    def _():
        _update(jnp.where(col < VOCAB, x_ref[...], NEG_INF), col)
        out_ref[...] = best_idx[...]


@functools.partial(
    pl.kernel,
    out_type=(jax.ShapeDtypeStruct((SC_ROWS * 16,), jnp.int32),
              jax.ShapeDtypeStruct((SC_ROWS * 16,), jnp.float32)),
    mesh=plsc.VectorSubcoreMesh(core_axis_name="c", subcore_axis_name="s"),
    scratch_types=[
        pltpu.VMEM((RPW, SCCH), jnp.float32),
        pltpu.VMEM((RPW, SCCH), jnp.float32),
        pltpu.VMEM((RPW * 16,), jnp.float32),
        pltpu.VMEM((RPW * 16,), jnp.int32),
        pltpu.VMEM((RPW * 16,), jnp.int32),
        pltpu.VMEM((RPW * 16,), jnp.float32),
        pltpu.SemaphoreType.DMA,
        pltpu.SemaphoreType.DMA,
    ],
)
def _sc_argmax(tp_hbm, outc_hbm, outv_hbm, buf0, buf1,
               bv_ref, bj_ref, resc_ref, resv_ref, sem0, sem1):
    wid = lax.axis_index("s") * 2 + lax.axis_index("c")
    lane = lax.iota(jnp.int32, 16)
    row0 = TC_ROWS + wid * RPW

    def init_row(r, _):
        bv_ref[pl.ds(r * 16, 16)] = jnp.full((16,), NEG_INF, jnp.float32)
        bj_ref[pl.ds(r * 16, 16)] = jnp.zeros((16,), jnp.int32)
        return 0

    lax.fori_loop(0, RPW, init_row, 0)

    def start(chunk, buf, sem, width=SCCH):
        off = pl.multiple_of(chunk * SCCH, 128)
        return pltpu.async_copy(
            tp_hbm.at[pl.ds(row0, RPW), pl.ds(off, width)],
            buf.at[:, pl.ds(0, width)], sem)

    def wait(buf, sem, width=SCCH):
        pltpu.make_async_copy(
            tp_hbm.at[pl.ds(row0, RPW), pl.ds(0, width)],
            buf.at[:, pl.ds(0, width)], sem).wait()

    def scan_chunk(buf, base, iters, unroll):
        def row_body(r, _):
            pv = bv_ref[pl.ds(r * 16, 16)]
            pj = bj_ref[pl.ds(r * 16, 16)]

            def body(j, carry):
                pv, pj = carry
                for u in range(unroll):
                    x = buf[r, pl.ds(j * (unroll * 16) + u * 16, 16)]
                    better = x > pv
                    pv = jnp.where(better, x, pv)
                    pj = jnp.where(
                        better,
                        jnp.full((16,), base + j * unroll + u, jnp.int32), pj)
                return pv, pj

            pv, pj = lax.fori_loop(0, iters // unroll, body, (pv, pj))
            bv_ref[pl.ds(r * 16, 16)] = pv
            bj_ref[pl.ds(r * 16, 16)] = pj
            return 0

        lax.fori_loop(0, RPW, row_body, 0)

    start(0, buf0, sem0)
    start(1, buf1, sem1)

    def pair_body(p, _):
        k = 2 * p
        wait(buf0, sem0)
        scan_chunk(buf0, k * ITERS_U, ITERS_U, 16)
        start(k + 2, buf0, sem0)
        wait(buf1, sem1)
        scan_chunk(buf1, (k + 1) * ITERS_U, ITERS_U, 16)

        @pl.when(p < (NUNI - 1) // 2 - 1)
        def _():
            start(k + 3, buf1, sem1)

        @pl.when(p == (NUNI - 1) // 2 - 1)
        def _():
            start(NUNI, buf1, sem1, width=SCTAIL)

        return 0

    lax.fori_loop(0, (NUNI - 1) // 2, pair_body, 0)

    wait(buf0, sem0)
    scan_chunk(buf0, (NUNI - 1) * ITERS_U, ITERS_U, 16)
    wait(buf1, sem1, width=SCTAIL)
    scan_chunk(buf1, NUNI * ITERS_U, ITERS_T, 8)

    for r in range(RPW):
        resc_ref[pl.ds(r * 16, 16)] = bj_ref[pl.ds(r * 16, 16)] * 16 + lane
        resv_ref[pl.ds(r * 16, 16)] = bv_ref[pl.ds(r * 16, 16)]
    pltpu.sync_copy(resc_ref, outc_hbm.at[pl.ds(wid * RPW * 16, RPW * 16)])
    pltpu.sync_copy(resv_ref, outv_hbm.at[pl.ds(wid * RPW * 16, RPW * 16)])


def _merge_body(tail_ref, vals_ref, cols_ref, out_ref):
    v = vals_ref[...]
    m = jnp.max(v, axis=1, keepdims=True)
    sci = jnp.min(jnp.where(v == m, cols_ref[...], BIG_I32),
                  axis=1, keepdims=True)
    tt = tail_ref[...][TC_ROWS:, :]
    col = jax.lax.broadcasted_iota(jnp.int32, (SC_ROWS, 128), 1) + SC_COLS
    ttm = jnp.where(col < VOCAB, tt, NEG_INF)
    tm = jnp.max(ttm, axis=1, keepdims=True)
    ti = jnp.min(jnp.where(ttm == tm, col, BIG_I32), axis=1, keepdims=True)
    better = tm > m
    out_ref[...] = jnp.where(better, ti, sci)


def _accept_body(ids_ref, draft_ref, out_ref):
    ids = ids_ref[...]
    draft = draft_ref[...]
    match = ids[:, :8] == draft
    j = jax.lax.broadcasted_iota(jnp.int32, (64, 8), 1)
    n = jnp.min(jnp.where(match, jnp.int32(8), j), axis=1, keepdims=True)
    p = jax.lax.broadcasted_iota(jnp.int32, (64, 9), 1)
    out_ref[...] = jnp.where(p <= n, ids, jnp.int32(-1))


@jax.jit
def kernel(draft_token_ids, target_probs):
    tc_idx = pl.pallas_call(
        _tc_argmax_body,
        grid=(NCHUNK,),
        in_specs=[pl.BlockSpec((TC_ROWS, CHUNK), lambda i: (0, i))],
        out_specs=pl.BlockSpec((TC_ROWS, 1), lambda i: (0, 0)),
        out_shape=jax.ShapeDtypeStruct((TC_ROWS, 1), jnp.int32),
        scratch_shapes=[
            pltpu.VMEM((TC_ROWS, 1), jnp.float32),
            pltpu.VMEM((TC_ROWS, 1), jnp.int32),
        ],
    )(target_probs)

    sc_cols, sc_vals = _sc_argmax(target_probs)
    sc_cols = sc_cols.reshape(SC_ROWS, 16)
    sc_vals = sc_vals.reshape(SC_ROWS, 16)

    merged = pl.pallas_call(
        _merge_body,
        grid=(1,),
        in_specs=[
            pl.BlockSpec((ROWS, 128), lambda i: (0, SC_COLS // 128)),
            pl.BlockSpec((SC_ROWS, 16), lambda i: (0, 0)),
            pl.BlockSpec((SC_ROWS, 16), lambda i: (0, 0)),
        ],
        out_specs=pl.BlockSpec((SC_ROWS, 1), lambda i: (0, 0)),
        out_shape=jax.ShapeDtypeStruct((SC_ROWS, 1), jnp.int32),
    )(target_probs, sc_vals, sc_cols)

    ids = jnp.concatenate(
        [tc_idx.reshape(TC_ROWS), merged.reshape(SC_ROWS)]).reshape(64, 9)
    out = pl.pallas_call(
        _accept_body,
        out_shape=jax.ShapeDtypeStruct((64, 9), jnp.int32),
    )(ids, draft_token_ids.astype(jnp.int32))
    return out.astype(jnp.int64)
